# Initial kernel scaffold; baseline (speedup 1.0000x reference)
import jax
import jax.numpy as jnp
from jax.experimental import pallas as pl


def kernel(logits, target):
    raise NotImplementedError("write your pallas kernel here")


# speedup vs baseline: 10.1010x; 10.1010x over previous
import functools

import jax
import jax.numpy as jnp
from jax.experimental import pallas as pl
from jax.experimental.pallas import tpu as pltpu

_C = 19
_BH = 64


def _body(x_ref, out_ref, acc_ref, *, nh, inv_n):
    n = pl.program_id(0)
    h = pl.program_id(1)

    x0 = x_ref[0, 0]
    m = x0
    amax = jnp.zeros(x0.shape, jnp.int32)
    s1 = x0
    s2 = x0 * x0
    for c in range(1, _C):
        xc = x_ref[0, c]
        gt = xc > m
        m = jnp.where(gt, xc, m)
        amax = jnp.where(gt, c, amax)
        s1 = s1 + xc
        s2 = s2 + xc * xc

    @pl.when(h == 0)
    def _():
        acc_ref[...] = jnp.zeros_like(acc_ref)

    for k in range(_C):
        mk = amax == k
        cnt_p = jnp.sum(mk.astype(jnp.float32), axis=0, keepdims=True)
        s1_p = jnp.sum(jnp.where(mk, s1, 0.0), axis=0, keepdims=True)
        s2_p = jnp.sum(jnp.where(mk, s2, 0.0), axis=0, keepdims=True)
        acc_ref[k : k + 1, :] += cnt_p
        acc_ref[_C + k : _C + k + 1, :] += s1_p
        acc_ref[2 * _C + k : 2 * _C + k + 1, :] += s2_p

    @pl.when(jnp.logical_and(n == 0, h == 0))
    def _():
        out_ref[0, 0] = 0.0

    @pl.when(h == nh - 1)
    def _():
        acc = acc_ref[...]
        cnt = jnp.sum(acc[0:_C], axis=1, keepdims=True) * float(_C)
        s1t = jnp.sum(acc[_C : 2 * _C], axis=1, keepdims=True)
        s2t = jnp.sum(acc[2 * _C : 3 * _C], axis=1, keepdims=True)
        norms = jnp.sqrt(s2t - s1t * s1t / cnt)
        out_ref[0, 0] += jnp.sum(norms) * inv_n


def kernel(logits, target):
    del target
    n, c, hh, w = logits.shape
    nh = hh // _BH
    out = pl.pallas_call(
        functools.partial(_body, nh=nh, inv_n=1.0 / n),
        grid=(n, nh),
        in_specs=[
            pl.BlockSpec((1, c, _BH, w), lambda i, j: (i, 0, j, 0)),
        ],
        out_specs=pl.BlockSpec(memory_space=pltpu.SMEM),
        out_shape=jax.ShapeDtypeStruct((1, 1), jnp.float32),
        scratch_shapes=[pltpu.VMEM((3 * _C, w), jnp.float32)],
    )(logits)
    return out[0, 0]
